# baseline (speedup 1.0000x reference)
import jax
import jax.numpy as jnp
from jax.experimental import pallas as pl
from jax.experimental.pallas import tpu as pltpu

_SIGMA = 0.1
_NCORES = 2
_CH = 512


def _round_up(v, m):
    return ((v + m - 1) // m) * m


def _make_kernel(bt, nc):
    def _kern(x_hbm, w_ref, nw_ref, b_ref, nb_ref, o_hbm,
              x_vm, weff_ref, beff_ref, o_vm, in_sem, out_sem):
        base = pl.program_id(0) * bt

        def in_copy(c):
            sl = pl.ds(c * _CH, _CH)
            return pltpu.make_async_copy(
                x_hbm.at[pl.ds(base + c * _CH, _CH), :],
                x_vm.at[sl, :], in_sem.at[c])

        def out_copy(c):
            sl = pl.ds(c * _CH, _CH)
            return pltpu.make_async_copy(
                o_vm.at[sl, :],
                o_hbm.at[pl.ds(base + c * _CH, _CH), :], out_sem.at[c])

        for c in range(nc):
            in_copy(c).start()

        weff_ref[...] = (w_ref[...] + _SIGMA * nw_ref[...]).astype(jnp.bfloat16)
        beff_ref[...] = b_ref[...] + _SIGMA * nb_ref[...]

        for c in range(nc):
            sl = pl.ds(c * _CH, _CH)
            in_copy(c).wait()
            o_vm[sl, :] = (
                jnp.dot(x_vm[sl, :].astype(jnp.bfloat16), weff_ref[...],
                        preferred_element_type=jnp.float32)
                + beff_ref[...]
            )
            out_copy(c).start()

        for c in range(nc):
            out_copy(c).wait()

    return _kern


def kernel(x, w_t, bias2d, noise_w_t, noise_b2d):
    B, K = x.shape
    Kw, N = w_t.shape
    assert K == Kw

    bt = _round_up(B, _CH * _NCORES) // _NCORES
    Bp = bt * _NCORES
    x_p = x if Bp == B else jnp.pad(x, ((0, Bp - B), (0, 0)))
    nc = bt // _CH

    out = pl.pallas_call(
        _make_kernel(bt, nc),
        grid=(_NCORES,),
        in_specs=[
            pl.BlockSpec(memory_space=pltpu.MemorySpace.HBM),
            pl.BlockSpec((K, N), lambda i: (0, 0)),
            pl.BlockSpec((K, N), lambda i: (0, 0)),
            pl.BlockSpec((1, N), lambda i: (0, 0)),
            pl.BlockSpec((1, N), lambda i: (0, 0)),
        ],
        out_specs=pl.BlockSpec(memory_space=pltpu.MemorySpace.HBM),
        out_shape=jax.ShapeDtypeStruct((Bp, N), jnp.float32),
        scratch_shapes=[
            pltpu.VMEM((bt, K), jnp.float32),
            pltpu.VMEM((K, N), jnp.bfloat16),
            pltpu.VMEM((1, N), jnp.float32),
            pltpu.VMEM((bt, N), jnp.float32),
            pltpu.SemaphoreType.DMA((nc,)),
            pltpu.SemaphoreType.DMA((nc,)),
        ],
        compiler_params=pltpu.CompilerParams(
            dimension_semantics=("parallel",),
            vmem_limit_bytes=48 << 20,
        ),
    )(x_p, w_t, noise_w_t, bias2d, noise_b2d)

    return out if Bp == B else out[:B]


# speedup vs baseline: 1.0659x; 1.0659x over previous
import jax
import jax.numpy as jnp
from jax.experimental import pallas as pl
from jax.experimental.pallas import tpu as pltpu

_SIGMA = 0.1
_NCORES = 2
_CH = 1024


def _round_up(v, m):
    return ((v + m - 1) // m) * m


def _make_kernel(bt, nc):
    def _kern(x_ref, w_ref, nw_ref, b_ref, nb_ref, o_hbm,
              weff_ref, beff_ref, o_vm, out_sem):
        base = pl.program_id(0) * bt

        weff_ref[...] = (w_ref[...] + _SIGMA * nw_ref[...]).astype(jnp.bfloat16)
        beff_ref[...] = b_ref[...] + _SIGMA * nb_ref[...]

        for c in range(nc):
            sl = pl.ds(c * _CH, _CH)
            o_vm[sl, :] = (
                jnp.dot(x_ref[sl, :].astype(jnp.bfloat16), weff_ref[...],
                        preferred_element_type=jnp.float32)
                + beff_ref[...]
            )
            pltpu.make_async_copy(
                o_vm.at[sl, :],
                o_hbm.at[pl.ds(base + c * _CH, _CH), :],
                out_sem.at[c]).start()

        for c in range(nc):
            pltpu.make_async_copy(
                o_vm.at[pl.ds(c * _CH, _CH), :],
                o_hbm.at[pl.ds(base + c * _CH, _CH), :],
                out_sem.at[c]).wait()

    return _kern


def kernel(x, w_t, bias2d, noise_w_t, noise_b2d):
    B, K = x.shape
    Kw, N = w_t.shape
    assert K == Kw

    bt = _round_up(B, _CH * _NCORES) // _NCORES
    Bp = bt * _NCORES
    x_p = x if Bp == B else jnp.pad(x, ((0, Bp - B), (0, 0)))
    nc = bt // _CH

    out = pl.pallas_call(
        _make_kernel(bt, nc),
        grid=(_NCORES,),
        in_specs=[
            pl.BlockSpec((bt, K), lambda i: (i, 0)),
            pl.BlockSpec((K, N), lambda i: (0, 0)),
            pl.BlockSpec((K, N), lambda i: (0, 0)),
            pl.BlockSpec((1, N), lambda i: (0, 0)),
            pl.BlockSpec((1, N), lambda i: (0, 0)),
        ],
        out_specs=pl.BlockSpec(memory_space=pltpu.MemorySpace.HBM),
        out_shape=jax.ShapeDtypeStruct((Bp, N), jnp.float32),
        scratch_shapes=[
            pltpu.VMEM((K, N), jnp.bfloat16),
            pltpu.VMEM((1, N), jnp.float32),
            pltpu.VMEM((bt, N), jnp.float32),
            pltpu.SemaphoreType.DMA((nc,)),
        ],
        compiler_params=pltpu.CompilerParams(
            dimension_semantics=("parallel",),
            vmem_limit_bytes=48 << 20,
        ),
    )(x_p, w_t, noise_w_t, bias2d, noise_b2d)

    return out if Bp == B else out[:B]


# speedup vs baseline: 1.1579x; 1.0863x over previous
import jax
import jax.numpy as jnp
from jax.experimental import pallas as pl
from jax.experimental.pallas import tpu as pltpu

_SIGMA = 0.1
_NCORES = 2
_CH = 256


def _round_up(v, m):
    return ((v + m - 1) // m) * m


def _make_kernel(bt, nc):
    def _kern(x_ref, w_ref, nw_ref, b_ref, nb_ref, o_hbm,
              weff_ref, beff_ref, o_vm, out_sem):
        base = pl.program_id(0) * bt

        weff_ref[...] = (w_ref[...] + _SIGMA * nw_ref[...]).astype(jnp.bfloat16)
        beff_ref[...] = b_ref[...] + _SIGMA * nb_ref[...]

        for c in range(nc):
            sl = pl.ds(c * _CH, _CH)
            o_vm[sl, :] = (
                jnp.dot(x_ref[sl, :].astype(jnp.bfloat16), weff_ref[...],
                        preferred_element_type=jnp.float32)
                + beff_ref[...]
            )
            pltpu.make_async_copy(
                o_vm.at[sl, :],
                o_hbm.at[pl.ds(base + c * _CH, _CH), :],
                out_sem.at[c]).start()

        for c in range(nc):
            pltpu.make_async_copy(
                o_vm.at[pl.ds(c * _CH, _CH), :],
                o_hbm.at[pl.ds(base + c * _CH, _CH), :],
                out_sem.at[c]).wait()

    return _kern


def kernel(x, w_t, bias2d, noise_w_t, noise_b2d):
    B, K = x.shape
    Kw, N = w_t.shape
    assert K == Kw

    bt = _round_up(B, _CH * _NCORES) // _NCORES
    Bp = bt * _NCORES
    x_p = x if Bp == B else jnp.pad(x, ((0, Bp - B), (0, 0)))
    nc = bt // _CH

    out = pl.pallas_call(
        _make_kernel(bt, nc),
        grid=(_NCORES,),
        in_specs=[
            pl.BlockSpec((bt, K), lambda i: (i, 0)),
            pl.BlockSpec((K, N), lambda i: (0, 0)),
            pl.BlockSpec((K, N), lambda i: (0, 0)),
            pl.BlockSpec((1, N), lambda i: (0, 0)),
            pl.BlockSpec((1, N), lambda i: (0, 0)),
        ],
        out_specs=pl.BlockSpec(memory_space=pltpu.MemorySpace.HBM),
        out_shape=jax.ShapeDtypeStruct((Bp, N), jnp.float32),
        scratch_shapes=[
            pltpu.VMEM((K, N), jnp.bfloat16),
            pltpu.VMEM((1, N), jnp.float32),
            pltpu.VMEM((bt, N), jnp.float32),
            pltpu.SemaphoreType.DMA((nc,)),
        ],
        compiler_params=pltpu.CompilerParams(
            dimension_semantics=("parallel",),
            vmem_limit_bytes=48 << 20,
        ),
    )(x_p, w_t, noise_w_t, bias2d, noise_b2d)

    return out if Bp == B else out[:B]
